# baseline (speedup 1.0000x reference)
import functools

import jax
import jax.numpy as jnp
from jax import lax
from jax.experimental import pallas as pl
from jax.experimental.pallas import tpu as pltpu
from jax.experimental.pallas import tpu_sc as plsc

DIM = 64
DIMOUT = 64
SZ = 512
N = 524288
TBL = SZ * SZ

NC = 2
NS = 16
L = 16
NW = NC * NS
PTS_PER_W = N // NW
P = 32
NCH = PTS_PER_W // P
NG = P // L
NCORNER = 12


_BH = 8


def _transform_body(xy_ref, xz_ref, yz_ref, w_ref, o0, o1, o2):
    zeros = jnp.zeros((SZ, DIMOUT), jnp.bfloat16)
    for p, (ref, out) in enumerate(((xy_ref, o0), (xz_ref, o1), (yz_ref, o2))):
        wp = w_ref[:, p * DIM:(p + 1) * DIM]
        for b in range(_BH):
            res = lax.dot_general(
                ref[:, b, :], wp, (((0,), (1,)), ((), ())),
                preferred_element_type=jnp.float32)
            out[b, :, 0:DIMOUT] = res.astype(jnp.bfloat16)
            out[b, :, DIMOUT:2 * DIMOUT] = zeros


def _transform(xy, xz, yz, lin_w):
    grid = (SZ // _BH,)
    in_spec = pl.BlockSpec((DIM, _BH, SZ), lambda i: (0, i, 0))
    w_spec = pl.BlockSpec((DIMOUT, 3 * DIM), lambda i: (0, 0))
    out_spec = pl.BlockSpec((_BH, SZ, 2 * DIMOUT), lambda i: (i, 0, 0))
    tbl_shape = jax.ShapeDtypeStruct((SZ, SZ, 2 * DIMOUT), jnp.bfloat16)
    return pl.pallas_call(
        _transform_body,
        grid=grid,
        in_specs=[in_spec, in_spec, in_spec, w_spec],
        out_specs=[out_spec, out_spec, out_spec],
        out_shape=[tbl_shape, tbl_shape, tbl_shape],
        compiler_params=pltpu.CompilerParams(
            dimension_semantics=("arbitrary",)),
    )(xy, xz, yz, lin_w)


def _axis_quantities(g):
    ix = ((g + 1.0) * float(SZ) - 1.0) * 0.5
    t = ix.astype(jnp.int32)
    tf = t.astype(jnp.float32)
    x0 = jnp.where(ix < tf, t - 1, t)
    x0f = x0.astype(jnp.float32)
    w1 = ix - x0f
    w0 = 1.0 - w1
    u0 = jnp.where((x0 >= 0) & (x0 <= SZ - 1), w0, 0.0)
    u1 = jnp.where((x0 >= -1) & (x0 <= SZ - 2), w1, 0.0)
    c0 = jnp.clip(x0, 0, SZ - 1)
    c1 = jnp.clip(x0 + 1, 0, SZ - 1)
    return c0, c1, u0, u1


def _sc_body(gx_hbm, gy_hbm, gz_hbm, t0_hbm, t1_hbm, t2_hbm, bias_hbm,
             out_hbm,
             cx0, cy0, cz0, cx1, cy1, cz1, idx0, idx1, w0, w1,
             rows0, rows1, ob0, ob1, biasb, biasp,
             xsem0, xsem1, gsem0, gsem1, osem0, osem1):
    cid = lax.axis_index("c")
    sid = lax.axis_index("s")
    wid = sid * NC + cid
    base = wid * PTS_PER_W
    tables = (t0_hbm, t1_hbm, t2_hbm)

    pltpu.sync_copy(bias_hbm, biasb)
    jv0 = lax.iota(jnp.int32, L)
    for m in range(DIMOUT // L):
        fvec = 32 * (m // 2) + 2 * jv0 + (m % 2)
        biasp[pl.ds(m * L, L)] = plsc.load_gather(biasb, [fvec])

    def stage_coords(g, cx, cy, cz, xsem):
        sl = pl.ds(base + g * P, P)
        pltpu.async_copy(gx_hbm.at[sl], cx, xsem)
        pltpu.async_copy(gy_hbm.at[sl], cy, xsem)
        pltpu.async_copy(gz_hbm.at[sl], cz, xsem)

    def wait_coords(cx, cy, cz, xsem):
        pltpu.make_async_copy(gx_hbm.at[pl.ds(0, P)], cx, xsem).wait()
        pltpu.make_async_copy(gx_hbm.at[pl.ds(0, P)], cy, xsem).wait()
        pltpu.make_async_copy(gx_hbm.at[pl.ds(0, P)], cz, xsem).wait()

    def fire(cx, cy, cz, idx2d, w2d, rows, gsem):
        def grp_body(grp, carry):
            sl = pl.ds(grp * L, L)
            ax = _axis_quantities(cx[sl])
            ay = _axis_quantities(cy[sl])
            az = _axis_quantities(cz[sl])
            c = 0
            for (A, B) in ((ax, ay), (ax, az), (ay, az)):
                b0 = B[0] * (2 * SZ)
                b1 = B[1] * (2 * SZ)
                for (brow, bw) in ((b0, B[2]), (b1, B[3])):
                    for i in (0, 1):
                        idx2d[c, sl] = brow + 2 * A[i]
                        w2d[c, sl] = A[2 + i] * bw
                        c += 1
            return carry

        lax.fori_loop(0, NG, grp_body, 0)
        for c in range(NCORNER):
            pltpu.async_copy(
                tables[c // 4].at[idx2d.at[c]], rows.at[c], gsem)

    def combine(g, w2d, rows, outb, gsem, osem, owait_pred):
        for c in range(NCORNER):
            pltpu.make_async_copy(
                t0_hbm.at[pl.ds(0, P)], rows.at[c], gsem).wait()

        @pl.when(owait_pred)
        def _():
            pltpu.make_async_copy(
                out_hbm.at[:, 0, :, pl.ds(0, P)], outb, osem).wait()

        jv = lax.iota(jnp.int32, L)
        sidx = []
        for h in range(DIMOUT // 32):
            for par in (0, 1):
                f = 32 * h + 2 * jv + par
                sidx.append((lax.shift_right_logical(f, 3),
                             jnp.bitwise_and(f, 7)))

        def grp_body(grp, carry):
            sl = pl.ds(grp * L, L)
            wvs = [w2d[c, sl] for c in range(NCORNER)]
            for lane in range(L):
                p = grp * L + lane
                pfull = jnp.full((L,), p, jnp.int32)
                accs = [biasp[pl.ds(m * L, L)]
                        for m in range(DIMOUT // L)]
                for c in range(NCORNER):
                    w = wvs[c][lane]
                    for h in range(DIMOUT // 32):
                        half = rows[c, p, pl.ds(h * 32, 32)]
                        ev, od = plsc.unpack(
                            half, format=plsc.PackFormat.INTERLEAVED)
                        accs[2 * h] = accs[2 * h] + w * ev
                        accs[2 * h + 1] = accs[2 * h + 1] + w * od
                for a, (bi, fi) in zip(accs, sidx):
                    plsc.store_scatter(outb, [bi, fi, pfull], a)
            return carry

        lax.fori_loop(0, NG, grp_body, 0)
        p0 = base + g * P
        pltpu.async_copy(
            outb, out_hbm.at[:, p0 // 128, :, pl.ds(p0 % 128, P)], osem)

    stage_coords(0, cx0, cy0, cz0, xsem0)
    stage_coords(1, cx1, cy1, cz1, xsem1)
    wait_coords(cx0, cy0, cz0, xsem0)
    fire(cx0, cy0, cz0, idx0, w0, rows0, gsem0)

    def body(h, carry):
        g0 = 2 * h
        g1 = g0 + 1
        wait_coords(cx1, cy1, cz1, xsem1)
        fire(cx1, cy1, cz1, idx1, w1, rows1, gsem1)

        @pl.when(g0 + 2 < NCH)
        def _():
            stage_coords(g0 + 2, cx0, cy0, cz0, xsem0)

        combine(g0, w0, rows0, ob0, gsem0, osem0, h >= 1)

        @pl.when(g0 + 2 < NCH)
        def _():
            wait_coords(cx0, cy0, cz0, xsem0)
            fire(cx0, cy0, cz0, idx0, w0, rows0, gsem0)

        @pl.when(g1 + 2 < NCH)
        def _():
            stage_coords(g1 + 2, cx1, cy1, cz1, xsem1)

        combine(g1, w1, rows1, ob1, gsem1, osem1, h >= 1)
        return carry

    lax.fori_loop(0, NCH // 2, body, 0)
    pltpu.make_async_copy(
        out_hbm.at[:, 0, :, pl.ds(0, P)], ob0, osem0).wait()
    pltpu.make_async_copy(
        out_hbm.at[:, 0, :, pl.ds(0, P)], ob1, osem1).wait()


@functools.partial(
    pl.kernel,
    out_type=jax.ShapeDtypeStruct((DIMOUT // 8, N // 128, 8, 128),
                                  jnp.float32),
    mesh=plsc.VectorSubcoreMesh(core_axis_name="c", subcore_axis_name="s"),
    compiler_params=pltpu.CompilerParams(
        use_tc_tiling_on_sc=False, needs_layout_passes=False),
    scratch_types=[
        pltpu.VMEM((P,), jnp.float32),
        pltpu.VMEM((P,), jnp.float32),
        pltpu.VMEM((P,), jnp.float32),
        pltpu.VMEM((P,), jnp.float32),
        pltpu.VMEM((P,), jnp.float32),
        pltpu.VMEM((P,), jnp.float32),
        pltpu.VMEM((NCORNER, P), jnp.int32),
        pltpu.VMEM((NCORNER, P), jnp.int32),
        pltpu.VMEM((NCORNER, P), jnp.float32),
        pltpu.VMEM((NCORNER, P), jnp.float32),
        pltpu.VMEM((NCORNER, P, DIMOUT), jnp.bfloat16),
        pltpu.VMEM((NCORNER, P, DIMOUT), jnp.bfloat16),
        pltpu.VMEM((DIMOUT // 8, 8, P), jnp.float32),
        pltpu.VMEM((DIMOUT // 8, 8, P), jnp.float32),
        pltpu.VMEM((DIMOUT,), jnp.float32),
        pltpu.VMEM((DIMOUT,), jnp.float32),
        pltpu.SemaphoreType.DMA,
        pltpu.SemaphoreType.DMA,
        pltpu.SemaphoreType.DMA,
        pltpu.SemaphoreType.DMA,
        pltpu.SemaphoreType.DMA,
        pltpu.SemaphoreType.DMA,
    ],
)
def _sc_sample(gx_hbm, gy_hbm, gz_hbm, t0_hbm, t1_hbm, t2_hbm, bias_hbm,
               out_hbm, *scratch):
    _sc_body(gx_hbm, gy_hbm, gz_hbm, t0_hbm, t1_hbm, t2_hbm, bias_hbm,
             out_hbm, *scratch)


def kernel(x, xy, xz, yz, lin_w, lin_b):
    gx = x[:, 0]
    gy = x[:, 1]
    gz = x[:, 2]
    t0, t1, t2 = _transform(xy, xz, yz, lin_w)
    t0 = t0.reshape(2 * TBL, DIMOUT)
    t1 = t1.reshape(2 * TBL, DIMOUT)
    t2 = t2.reshape(2 * TBL, DIMOUT)
    out4 = _sc_sample(gx, gy, gz, t0, t1, t2, lin_b)
    return out4.transpose(1, 3, 0, 2).reshape(N, DIMOUT)


# speedup vs baseline: 1.3130x; 1.3130x over previous
import functools

import jax
import jax.numpy as jnp
from jax import lax
from jax.experimental import pallas as pl
from jax.experimental.pallas import tpu as pltpu
from jax.experimental.pallas import tpu_sc as plsc

DIM = 64
DIMOUT = 64
SZ = 512
N = 524288
TBL = SZ * SZ

NC = 2
NS = 16
L = 16
NW = NC * NS
PTS_PER_W = N // NW
P = 32
NCH = PTS_PER_W // P
NG = P // L
NCORNER = 12


_BH = 8


def _transform_body(xy_ref, xz_ref, yz_ref, w_ref, o0, o1, o2):
    for p, (ref, out) in enumerate(((xy_ref, o0), (xz_ref, o1), (yz_ref, o2))):
        we = w_ref[0:DIMOUT // 2, p * DIM:(p + 1) * DIM]
        wo = w_ref[DIMOUT // 2:DIMOUT, p * DIM:(p + 1) * DIM]
        for b in range(_BH):
            blk = ref[:, b, :]
            res_e = lax.dot_general(
                blk, we, (((0,), (1,)), ((), ())),
                preferred_element_type=jnp.float32)
            res_o = lax.dot_general(
                blk, wo, (((0,), (1,)), ((), ())),
                preferred_element_type=jnp.float32)
            ue = lax.bitcast_convert_type(
                res_e.astype(jnp.bfloat16), jnp.uint16).astype(jnp.uint32)
            uo = lax.bitcast_convert_type(
                res_o.astype(jnp.bfloat16), jnp.uint16).astype(jnp.uint32)
            packed = lax.bitcast_convert_type(
                ue | (uo << 16), jnp.float32)
            out[b, :, 0:DIMOUT // 2] = packed


def _transform(xy, xz, yz, lin_w):
    grid = (SZ // _BH,)
    in_spec = pl.BlockSpec((DIM, _BH, SZ), lambda i: (0, i, 0))
    w_spec = pl.BlockSpec((DIMOUT, 3 * DIM), lambda i: (0, 0))
    out_spec = pl.BlockSpec((_BH, SZ, 2 * DIMOUT), lambda i: (i, 0, 0))
    tbl_shape = jax.ShapeDtypeStruct((SZ, SZ, 2 * DIMOUT), jnp.float32)
    return pl.pallas_call(
        _transform_body,
        grid=grid,
        in_specs=[in_spec, in_spec, in_spec, w_spec],
        out_specs=[out_spec, out_spec, out_spec],
        out_shape=[tbl_shape, tbl_shape, tbl_shape],
        compiler_params=pltpu.CompilerParams(
            dimension_semantics=("arbitrary",)),
    )(xy, xz, yz, lin_w)


def _axis_quantities(g):
    ix = ((g + 1.0) * float(SZ) - 1.0) * 0.5
    t = ix.astype(jnp.int32)
    tf = t.astype(jnp.float32)
    x0 = jnp.where(ix < tf, t - 1, t)
    x0f = x0.astype(jnp.float32)
    w1 = ix - x0f
    w0 = 1.0 - w1
    u0 = jnp.where((x0 >= 0) & (x0 <= SZ - 1), w0, 0.0)
    u1 = jnp.where((x0 >= -1) & (x0 <= SZ - 2), w1, 0.0)
    c0 = jnp.clip(x0, 0, SZ - 1)
    c1 = jnp.clip(x0 + 1, 0, SZ - 1)
    return c0, c1, u0, u1


def _sc_body(gx_hbm, gy_hbm, gz_hbm, t0_hbm, t1_hbm, t2_hbm, bias_hbm,
             out_hbm,
             cx0, cy0, cz0, cx1, cy1, cz1, idx0, idx1, w0, w1,
             rows0, rows1, ob0, ob1, biasb, biasp,
             xsem0, xsem1, gsem0, gsem1, osem0, osem1):
    cid = lax.axis_index("c")
    sid = lax.axis_index("s")
    wid = sid * NC + cid
    base = wid * PTS_PER_W
    tables = (t0_hbm, t1_hbm, t2_hbm)

    pltpu.sync_copy(bias_hbm, biasb)
    jv0 = lax.iota(jnp.int32, L)
    for m in range(DIMOUT // L):
        fvec = 32 * (m // 2) + 2 * jv0 + (m % 2)
        biasp[pl.ds(m * L, L)] = plsc.load_gather(biasb, [fvec])

    def stage_coords(g, cx, cy, cz, xsem):
        sl = pl.ds(base + g * P, P)
        pltpu.async_copy(gx_hbm.at[sl], cx, xsem)
        pltpu.async_copy(gy_hbm.at[sl], cy, xsem)
        pltpu.async_copy(gz_hbm.at[sl], cz, xsem)

    def wait_coords(cx, cy, cz, xsem):
        pltpu.make_async_copy(gx_hbm.at[pl.ds(0, P)], cx, xsem).wait()
        pltpu.make_async_copy(gx_hbm.at[pl.ds(0, P)], cy, xsem).wait()
        pltpu.make_async_copy(gx_hbm.at[pl.ds(0, P)], cz, xsem).wait()

    def fire(cx, cy, cz, idx2d, w2d, rows, gsem):
        def grp_body(grp, carry):
            sl = pl.ds(grp * L, L)
            ax = _axis_quantities(cx[sl])
            ay = _axis_quantities(cy[sl])
            az = _axis_quantities(cz[sl])
            c = 0
            for (A, B) in ((ax, ay), (ax, az), (ay, az)):
                b0 = B[0] * (4 * SZ)
                b1 = B[1] * (4 * SZ)
                for (brow, bw) in ((b0, B[2]), (b1, B[3])):
                    for i in (0, 1):
                        idx2d[c, sl] = brow + 4 * A[i]
                        w2d[c, sl] = A[2 + i] * bw
                        c += 1
            return carry

        lax.fori_loop(0, NG, grp_body, 0)
        for c in range(NCORNER):
            pltpu.async_copy(
                tables[c // 4].at[idx2d.at[c]], rows.at[c], gsem)

    def combine(g, w2d, rows, outb, gsem, osem, owait_pred):
        for c in range(NCORNER):
            pltpu.make_async_copy(
                t0_hbm.at[pl.ds(0, P)], rows.at[c], gsem).wait()

        @pl.when(owait_pred)
        def _():
            pltpu.make_async_copy(
                out_hbm.at[:, 0, :, pl.ds(0, P)], outb, osem).wait()

        jv = lax.iota(jnp.int32, L)
        sidx = []
        for h in range(DIMOUT // 32):
            for par in (0, 1):
                f = 32 * h + 2 * jv + par
                sidx.append((lax.shift_right_logical(f, 3),
                             jnp.bitwise_and(f, 7)))

        def grp_body(grp, carry):
            sl = pl.ds(grp * L, L)
            wvs = [w2d[c, sl] for c in range(NCORNER)]
            for lane in range(L):
                p = grp * L + lane
                pfull = jnp.full((L,), p, jnp.int32)
                accs = [biasp[pl.ds(m * L, L)]
                        for m in range(DIMOUT // L)]
                for c in range(NCORNER):
                    w = wvs[c][lane]
                    for h in range(DIMOUT // 32):
                        half = plsc.bitcast(
                            rows[c, p, pl.ds(h * L, L)], jnp.bfloat16)
                        ev, od = plsc.unpack(
                            half, format=plsc.PackFormat.INTERLEAVED)
                        accs[2 * h] = accs[2 * h] + w * ev
                        accs[2 * h + 1] = accs[2 * h + 1] + w * od
                for a, (bi, fi) in zip(accs, sidx):
                    plsc.store_scatter(outb, [bi, fi, pfull], a)
            return carry

        lax.fori_loop(0, NG, grp_body, 0)
        p0 = base + g * P
        pltpu.async_copy(
            outb, out_hbm.at[:, p0 // 128, :, pl.ds(p0 % 128, P)], osem)

    stage_coords(0, cx0, cy0, cz0, xsem0)
    stage_coords(1, cx1, cy1, cz1, xsem1)
    wait_coords(cx0, cy0, cz0, xsem0)
    fire(cx0, cy0, cz0, idx0, w0, rows0, gsem0)

    def body(h, carry):
        g0 = 2 * h
        g1 = g0 + 1
        wait_coords(cx1, cy1, cz1, xsem1)
        fire(cx1, cy1, cz1, idx1, w1, rows1, gsem1)

        @pl.when(g0 + 2 < NCH)
        def _():
            stage_coords(g0 + 2, cx0, cy0, cz0, xsem0)

        combine(g0, w0, rows0, ob0, gsem0, osem0, h >= 1)

        @pl.when(g0 + 2 < NCH)
        def _():
            wait_coords(cx0, cy0, cz0, xsem0)
            fire(cx0, cy0, cz0, idx0, w0, rows0, gsem0)

        @pl.when(g1 + 2 < NCH)
        def _():
            stage_coords(g1 + 2, cx1, cy1, cz1, xsem1)

        combine(g1, w1, rows1, ob1, gsem1, osem1, h >= 1)
        return carry

    lax.fori_loop(0, NCH // 2, body, 0)
    pltpu.make_async_copy(
        out_hbm.at[:, 0, :, pl.ds(0, P)], ob0, osem0).wait()
    pltpu.make_async_copy(
        out_hbm.at[:, 0, :, pl.ds(0, P)], ob1, osem1).wait()


@functools.partial(
    pl.kernel,
    out_type=jax.ShapeDtypeStruct((DIMOUT // 8, N // 128, 8, 128),
                                  jnp.float32),
    mesh=plsc.VectorSubcoreMesh(core_axis_name="c", subcore_axis_name="s"),
    compiler_params=pltpu.CompilerParams(
        use_tc_tiling_on_sc=False, needs_layout_passes=False),
    scratch_types=[
        pltpu.VMEM((P,), jnp.float32),
        pltpu.VMEM((P,), jnp.float32),
        pltpu.VMEM((P,), jnp.float32),
        pltpu.VMEM((P,), jnp.float32),
        pltpu.VMEM((P,), jnp.float32),
        pltpu.VMEM((P,), jnp.float32),
        pltpu.VMEM((NCORNER, P), jnp.int32),
        pltpu.VMEM((NCORNER, P), jnp.int32),
        pltpu.VMEM((NCORNER, P), jnp.float32),
        pltpu.VMEM((NCORNER, P), jnp.float32),
        pltpu.VMEM((NCORNER, P, DIMOUT // 2), jnp.float32),
        pltpu.VMEM((NCORNER, P, DIMOUT // 2), jnp.float32),
        pltpu.VMEM((DIMOUT // 8, 8, P), jnp.float32),
        pltpu.VMEM((DIMOUT // 8, 8, P), jnp.float32),
        pltpu.VMEM((DIMOUT,), jnp.float32),
        pltpu.VMEM((DIMOUT,), jnp.float32),
        pltpu.SemaphoreType.DMA,
        pltpu.SemaphoreType.DMA,
        pltpu.SemaphoreType.DMA,
        pltpu.SemaphoreType.DMA,
        pltpu.SemaphoreType.DMA,
        pltpu.SemaphoreType.DMA,
    ],
)
def _sc_sample(gx_hbm, gy_hbm, gz_hbm, t0_hbm, t1_hbm, t2_hbm, bias_hbm,
               out_hbm, *scratch):
    _sc_body(gx_hbm, gy_hbm, gz_hbm, t0_hbm, t1_hbm, t2_hbm, bias_hbm,
             out_hbm, *scratch)


def kernel(x, xy, xz, yz, lin_w, lin_b):
    gx = x[:, 0]
    gy = x[:, 1]
    gz = x[:, 2]
    wperm = jnp.concatenate([lin_w[0::2, :], lin_w[1::2, :]], axis=0)
    t0, t1, t2 = _transform(xy, xz, yz, wperm)
    t0 = t0.reshape(4 * TBL, DIMOUT // 2)
    t1 = t1.reshape(4 * TBL, DIMOUT // 2)
    t2 = t2.reshape(4 * TBL, DIMOUT // 2)
    out4 = _sc_sample(gx, gy, gz, t0, t1, t2, lin_b)
    return out4.transpose(1, 3, 0, 2).reshape(N, DIMOUT)


# speedup vs baseline: 1.5828x; 1.2055x over previous
import functools

import jax
import jax.numpy as jnp
from jax import lax
from jax.experimental import pallas as pl
from jax.experimental.pallas import tpu as pltpu
from jax.experimental.pallas import tpu_sc as plsc

DIM = 64
DIMOUT = 64
SZ = 512
N = 524288
TBL = SZ * SZ

NC = 2
NS = 16
L = 16
NW = NC * NS
PTS_PER_W = N // NW
P = 64
NCH = PTS_PER_W // P
NG = P // L
NCORNER = 12


_BH = 8


def _transform_body(xy_ref, xz_ref, yz_ref, w_ref, o0, o1, o2):
    for p, (ref, out) in enumerate(((xy_ref, o0), (xz_ref, o1), (yz_ref, o2))):
        we = w_ref[0:DIMOUT // 2, p * DIM:(p + 1) * DIM]
        wo = w_ref[DIMOUT // 2:DIMOUT, p * DIM:(p + 1) * DIM]
        for b in range(_BH):
            blk = ref[:, b, :]
            res_e = lax.dot_general(
                blk, we, (((0,), (1,)), ((), ())),
                preferred_element_type=jnp.float32)
            res_o = lax.dot_general(
                blk, wo, (((0,), (1,)), ((), ())),
                preferred_element_type=jnp.float32)
            ue = lax.bitcast_convert_type(
                res_e.astype(jnp.bfloat16), jnp.uint16).astype(jnp.uint32)
            uo = lax.bitcast_convert_type(
                res_o.astype(jnp.bfloat16), jnp.uint16).astype(jnp.uint32)
            packed = lax.bitcast_convert_type(
                ue | (uo << 16), jnp.float32)
            out[b, :, 0:DIMOUT // 2] = packed


def _transform(xy, xz, yz, lin_w):
    grid = (SZ // _BH,)
    in_spec = pl.BlockSpec((DIM, _BH, SZ), lambda i: (0, i, 0))
    w_spec = pl.BlockSpec((DIMOUT, 3 * DIM), lambda i: (0, 0))
    out_spec = pl.BlockSpec((_BH, SZ, 2 * DIMOUT), lambda i: (i, 0, 0))
    tbl_shape = jax.ShapeDtypeStruct((SZ, SZ, 2 * DIMOUT), jnp.float32)
    return pl.pallas_call(
        _transform_body,
        grid=grid,
        in_specs=[in_spec, in_spec, in_spec, w_spec],
        out_specs=[out_spec, out_spec, out_spec],
        out_shape=[tbl_shape, tbl_shape, tbl_shape],
        compiler_params=pltpu.CompilerParams(
            dimension_semantics=("arbitrary",)),
    )(xy, xz, yz, lin_w)


def _axis_quantities(g):
    ix = ((g + 1.0) * float(SZ) - 1.0) * 0.5
    t = ix.astype(jnp.int32)
    tf = t.astype(jnp.float32)
    x0 = jnp.where(ix < tf, t - 1, t)
    x0f = x0.astype(jnp.float32)
    w1 = ix - x0f
    w0 = 1.0 - w1
    u0 = jnp.where((x0 >= 0) & (x0 <= SZ - 1), w0, 0.0)
    u1 = jnp.where((x0 >= -1) & (x0 <= SZ - 2), w1, 0.0)
    c0 = jnp.clip(x0, 0, SZ - 1)
    c1 = jnp.clip(x0 + 1, 0, SZ - 1)
    return c0, c1, u0, u1


def _sc_body(gx_hbm, gy_hbm, gz_hbm, t0_hbm, t1_hbm, t2_hbm, bias_hbm,
             out_hbm,
             cx0, cy0, cz0, cx1, cy1, cz1, idx0, idx1, w0, w1,
             rows0, rows1, ob0, ob1, biasb, biasp,
             xsem0, xsem1, gsem0, gsem1, osem0, osem1):
    cid = lax.axis_index("c")
    sid = lax.axis_index("s")
    wid = sid * NC + cid
    base = wid * PTS_PER_W
    tables = (t0_hbm, t1_hbm, t2_hbm)

    pltpu.sync_copy(bias_hbm, biasb)
    jv0 = lax.iota(jnp.int32, L)
    for m in range(DIMOUT // L):
        fvec = 32 * (m // 2) + 2 * jv0 + (m % 2)
        biasp[pl.ds(m * L, L)] = plsc.load_gather(biasb, [fvec])

    def stage_coords(g, cx, cy, cz, xsem):
        sl = pl.ds(base + g * P, P)
        pltpu.async_copy(gx_hbm.at[sl], cx, xsem)
        pltpu.async_copy(gy_hbm.at[sl], cy, xsem)
        pltpu.async_copy(gz_hbm.at[sl], cz, xsem)

    def wait_coords(cx, cy, cz, xsem):
        pltpu.make_async_copy(gx_hbm.at[pl.ds(0, P)], cx, xsem).wait()
        pltpu.make_async_copy(gx_hbm.at[pl.ds(0, P)], cy, xsem).wait()
        pltpu.make_async_copy(gx_hbm.at[pl.ds(0, P)], cz, xsem).wait()

    def fire(cx, cy, cz, idx2d, w2d, rows, gsem):
        def grp_body(grp, carry):
            sl = pl.ds(grp * L, L)
            ax = _axis_quantities(cx[sl])
            ay = _axis_quantities(cy[sl])
            az = _axis_quantities(cz[sl])
            c = 0
            for (A, B) in ((ax, ay), (ax, az), (ay, az)):
                b0 = B[0] * (4 * SZ)
                b1 = B[1] * (4 * SZ)
                for (brow, bw) in ((b0, B[2]), (b1, B[3])):
                    for i in (0, 1):
                        idx2d[c, sl] = brow + 4 * A[i]
                        w2d[c, sl] = A[2 + i] * bw
                        c += 1
            return carry

        lax.fori_loop(0, NG, grp_body, 0)
        for c in range(NCORNER):
            pltpu.async_copy(
                tables[c // 4].at[idx2d.at[c]], rows.at[c], gsem)

    def combine(g, w2d, rows, outb, gsem, osem, owait_pred):
        for c in range(NCORNER):
            pltpu.make_async_copy(
                t0_hbm.at[pl.ds(0, P)], rows.at[c], gsem).wait()

        @pl.when(owait_pred)
        def _():
            pltpu.make_async_copy(
                out_hbm.at[:, 0, :, pl.ds(0, P)], outb, osem).wait()

        jv = lax.iota(jnp.int32, L)
        sidx = []
        for h in range(DIMOUT // 32):
            for par in (0, 1):
                f = 32 * h + 2 * jv + par
                sidx.append((lax.shift_right_logical(f, 3),
                             jnp.bitwise_and(f, 7)))

        def grp_body(grp, carry):
            sl = pl.ds(grp * L, L)
            wvs = [w2d[c, sl] for c in range(NCORNER)]
            for lane in range(L):
                p = grp * L + lane
                pfull = jnp.full((L,), p, jnp.int32)
                accs = [biasp[pl.ds(m * L, L)]
                        for m in range(DIMOUT // L)]
                for c in range(NCORNER):
                    w = wvs[c][lane]
                    for h in range(DIMOUT // 32):
                        half = plsc.bitcast(
                            rows[c, p, pl.ds(h * L, L)], jnp.bfloat16)
                        ev, od = plsc.unpack(
                            half, format=plsc.PackFormat.INTERLEAVED)
                        accs[2 * h] = accs[2 * h] + w * ev
                        accs[2 * h + 1] = accs[2 * h + 1] + w * od
                for a, (bi, fi) in zip(accs, sidx):
                    plsc.store_scatter(outb, [bi, fi, pfull], a)
            return carry

        lax.fori_loop(0, NG, grp_body, 0)
        p0 = base + g * P
        pltpu.async_copy(
            outb, out_hbm.at[:, p0 // 128, :, pl.ds(p0 % 128, P)], osem)

    stage_coords(0, cx0, cy0, cz0, xsem0)
    stage_coords(1, cx1, cy1, cz1, xsem1)
    wait_coords(cx0, cy0, cz0, xsem0)
    fire(cx0, cy0, cz0, idx0, w0, rows0, gsem0)

    def body(h, carry):
        g0 = 2 * h
        g1 = g0 + 1
        wait_coords(cx1, cy1, cz1, xsem1)
        fire(cx1, cy1, cz1, idx1, w1, rows1, gsem1)

        @pl.when(g0 + 2 < NCH)
        def _():
            stage_coords(g0 + 2, cx0, cy0, cz0, xsem0)

        combine(g0, w0, rows0, ob0, gsem0, osem0, h >= 1)

        @pl.when(g0 + 2 < NCH)
        def _():
            wait_coords(cx0, cy0, cz0, xsem0)
            fire(cx0, cy0, cz0, idx0, w0, rows0, gsem0)

        @pl.when(g1 + 2 < NCH)
        def _():
            stage_coords(g1 + 2, cx1, cy1, cz1, xsem1)

        combine(g1, w1, rows1, ob1, gsem1, osem1, h >= 1)
        return carry

    lax.fori_loop(0, NCH // 2, body, 0)
    pltpu.make_async_copy(
        out_hbm.at[:, 0, :, pl.ds(0, P)], ob0, osem0).wait()
    pltpu.make_async_copy(
        out_hbm.at[:, 0, :, pl.ds(0, P)], ob1, osem1).wait()


@functools.partial(
    pl.kernel,
    out_type=jax.ShapeDtypeStruct((DIMOUT // 8, N // 128, 8, 128),
                                  jnp.float32),
    mesh=plsc.VectorSubcoreMesh(core_axis_name="c", subcore_axis_name="s"),
    compiler_params=pltpu.CompilerParams(
        use_tc_tiling_on_sc=False, needs_layout_passes=False),
    scratch_types=[
        pltpu.VMEM((P,), jnp.float32),
        pltpu.VMEM((P,), jnp.float32),
        pltpu.VMEM((P,), jnp.float32),
        pltpu.VMEM((P,), jnp.float32),
        pltpu.VMEM((P,), jnp.float32),
        pltpu.VMEM((P,), jnp.float32),
        pltpu.VMEM((NCORNER, P), jnp.int32),
        pltpu.VMEM((NCORNER, P), jnp.int32),
        pltpu.VMEM((NCORNER, P), jnp.float32),
        pltpu.VMEM((NCORNER, P), jnp.float32),
        pltpu.VMEM((NCORNER, P, DIMOUT // 2), jnp.float32),
        pltpu.VMEM((NCORNER, P, DIMOUT // 2), jnp.float32),
        pltpu.VMEM((DIMOUT // 8, 8, P), jnp.float32),
        pltpu.VMEM((DIMOUT // 8, 8, P), jnp.float32),
        pltpu.VMEM((DIMOUT,), jnp.float32),
        pltpu.VMEM((DIMOUT,), jnp.float32),
        pltpu.SemaphoreType.DMA,
        pltpu.SemaphoreType.DMA,
        pltpu.SemaphoreType.DMA,
        pltpu.SemaphoreType.DMA,
        pltpu.SemaphoreType.DMA,
        pltpu.SemaphoreType.DMA,
    ],
)
def _sc_sample(gx_hbm, gy_hbm, gz_hbm, t0_hbm, t1_hbm, t2_hbm, bias_hbm,
               out_hbm, *scratch):
    _sc_body(gx_hbm, gy_hbm, gz_hbm, t0_hbm, t1_hbm, t2_hbm, bias_hbm,
             out_hbm, *scratch)


def kernel(x, xy, xz, yz, lin_w, lin_b):
    gx = x[:, 0]
    gy = x[:, 1]
    gz = x[:, 2]
    wperm = jnp.concatenate([lin_w[0::2, :], lin_w[1::2, :]], axis=0)
    t0, t1, t2 = _transform(xy, xz, yz, wperm)
    t0 = t0.reshape(4 * TBL, DIMOUT // 2)
    t1 = t1.reshape(4 * TBL, DIMOUT // 2)
    t2 = t2.reshape(4 * TBL, DIMOUT // 2)
    out4 = _sc_sample(gx, gy, gz, t0, t1, t2, lin_b)
    return out4.transpose(1, 3, 0, 2).reshape(N, DIMOUT)


# speedup vs baseline: 1.6373x; 1.0344x over previous
import functools

import jax
import jax.numpy as jnp
from jax import lax
from jax.experimental import pallas as pl
from jax.experimental.pallas import tpu as pltpu
from jax.experimental.pallas import tpu_sc as plsc

DIM = 64
DIMOUT = 64
SZ = 512
N = 524288
TBL = SZ * SZ

NC = 2
NS = 16
L = 16
NW = NC * NS
PTS_PER_W = N // NW
P = 128
NCH = PTS_PER_W // P
NG = P // L
NCORNER = 12


_BH = 8


def _transform_body(xy_ref, xz_ref, yz_ref, w_ref, o0, o1, o2):
    for p, (ref, out) in enumerate(((xy_ref, o0), (xz_ref, o1), (yz_ref, o2))):
        we = w_ref[0:DIMOUT // 2, p * DIM:(p + 1) * DIM]
        wo = w_ref[DIMOUT // 2:DIMOUT, p * DIM:(p + 1) * DIM]
        for b in range(_BH):
            blk = ref[:, b, :]
            res_e = lax.dot_general(
                blk, we, (((0,), (1,)), ((), ())),
                preferred_element_type=jnp.float32)
            res_o = lax.dot_general(
                blk, wo, (((0,), (1,)), ((), ())),
                preferred_element_type=jnp.float32)
            ue = lax.bitcast_convert_type(
                res_e.astype(jnp.bfloat16), jnp.uint16).astype(jnp.uint32)
            uo = lax.bitcast_convert_type(
                res_o.astype(jnp.bfloat16), jnp.uint16).astype(jnp.uint32)
            packed = lax.bitcast_convert_type(
                ue | (uo << 16), jnp.float32)
            out[b, :, 0:DIMOUT // 2] = packed


def _transform(xy, xz, yz, lin_w):
    grid = (SZ // _BH,)
    in_spec = pl.BlockSpec((DIM, _BH, SZ), lambda i: (0, i, 0))
    w_spec = pl.BlockSpec((DIMOUT, 3 * DIM), lambda i: (0, 0))
    out_spec = pl.BlockSpec((_BH, SZ, 2 * DIMOUT), lambda i: (i, 0, 0))
    tbl_shape = jax.ShapeDtypeStruct((SZ, SZ, 2 * DIMOUT), jnp.float32)
    return pl.pallas_call(
        _transform_body,
        grid=grid,
        in_specs=[in_spec, in_spec, in_spec, w_spec],
        out_specs=[out_spec, out_spec, out_spec],
        out_shape=[tbl_shape, tbl_shape, tbl_shape],
        compiler_params=pltpu.CompilerParams(
            dimension_semantics=("arbitrary",)),
    )(xy, xz, yz, lin_w)


def _axis_quantities(g):
    ix = ((g + 1.0) * float(SZ) - 1.0) * 0.5
    t = ix.astype(jnp.int32)
    tf = t.astype(jnp.float32)
    x0 = jnp.where(ix < tf, t - 1, t)
    x0f = x0.astype(jnp.float32)
    w1 = ix - x0f
    w0 = 1.0 - w1
    u0 = jnp.where((x0 >= 0) & (x0 <= SZ - 1), w0, 0.0)
    u1 = jnp.where((x0 >= -1) & (x0 <= SZ - 2), w1, 0.0)
    c0 = jnp.clip(x0, 0, SZ - 1)
    c1 = jnp.clip(x0 + 1, 0, SZ - 1)
    return c0, c1, u0, u1


def _sc_body(gx_hbm, gy_hbm, gz_hbm, t0_hbm, t1_hbm, t2_hbm, bias_hbm,
             out_hbm,
             cx0, cy0, cz0, cx1, cy1, cz1, idx0, idx1, w0, w1,
             rows0, rows1, ob0, ob1, biasb, biasp,
             xsem0, xsem1, gsem0, gsem1, osem0, osem1):
    cid = lax.axis_index("c")
    sid = lax.axis_index("s")
    wid = sid * NC + cid
    base = wid * PTS_PER_W
    tables = (t0_hbm, t1_hbm, t2_hbm)

    pltpu.sync_copy(bias_hbm, biasb)
    jv0 = lax.iota(jnp.int32, L)
    for m in range(DIMOUT // L):
        fvec = 32 * (m // 2) + 2 * jv0 + (m % 2)
        biasp[pl.ds(m * L, L)] = plsc.load_gather(biasb, [fvec])

    def stage_coords(g, cx, cy, cz, xsem):
        sl = pl.ds(base + g * P, P)
        pltpu.async_copy(gx_hbm.at[sl], cx, xsem)
        pltpu.async_copy(gy_hbm.at[sl], cy, xsem)
        pltpu.async_copy(gz_hbm.at[sl], cz, xsem)

    def wait_coords(cx, cy, cz, xsem):
        pltpu.make_async_copy(gx_hbm.at[pl.ds(0, P)], cx, xsem).wait()
        pltpu.make_async_copy(gx_hbm.at[pl.ds(0, P)], cy, xsem).wait()
        pltpu.make_async_copy(gx_hbm.at[pl.ds(0, P)], cz, xsem).wait()

    def fire(cx, cy, cz, idx2d, w2d, rows, gsem):
        def grp_body(grp, carry):
            sl = pl.ds(grp * L, L)
            ax = _axis_quantities(cx[sl])
            ay = _axis_quantities(cy[sl])
            az = _axis_quantities(cz[sl])
            c = 0
            for (A, B) in ((ax, ay), (ax, az), (ay, az)):
                b0 = B[0] * (4 * SZ)
                b1 = B[1] * (4 * SZ)
                for (brow, bw) in ((b0, B[2]), (b1, B[3])):
                    for i in (0, 1):
                        idx2d[c, sl] = brow + 4 * A[i]
                        w2d[c, sl] = A[2 + i] * bw
                        c += 1
            return carry

        lax.fori_loop(0, NG, grp_body, 0)
        for c in range(NCORNER):
            pltpu.async_copy(
                tables[c // 4].at[idx2d.at[c]], rows.at[c], gsem)

    def combine(g, w2d, rows, outb, gsem, osem, owait_pred):
        for c in range(NCORNER):
            pltpu.make_async_copy(
                t0_hbm.at[pl.ds(0, P)], rows.at[c], gsem).wait()

        @pl.when(owait_pred)
        def _():
            pltpu.make_async_copy(
                out_hbm.at[:, 0, :, pl.ds(0, P)], outb, osem).wait()

        jv = lax.iota(jnp.int32, L)
        sidx = []
        for h in range(DIMOUT // 32):
            for par in (0, 1):
                f = 32 * h + 2 * jv + par
                sidx.append((lax.shift_right_logical(f, 3),
                             jnp.bitwise_and(f, 7)))

        def grp_body(grp, carry):
            sl = pl.ds(grp * L, L)
            wvs = [w2d[c, sl] for c in range(NCORNER)]
            for lane in range(L):
                p = grp * L + lane
                pfull = jnp.full((L,), p, jnp.int32)
                accs = [biasp[pl.ds(m * L, L)]
                        for m in range(DIMOUT // L)]
                for c in range(NCORNER):
                    w = wvs[c][lane]
                    for h in range(DIMOUT // 32):
                        half = plsc.bitcast(
                            rows[c, p, pl.ds(h * L, L)], jnp.bfloat16)
                        ev, od = plsc.unpack(
                            half, format=plsc.PackFormat.INTERLEAVED)
                        accs[2 * h] = accs[2 * h] + w * ev
                        accs[2 * h + 1] = accs[2 * h + 1] + w * od
                for a, (bi, fi) in zip(accs, sidx):
                    plsc.store_scatter(outb, [bi, fi, pfull], a)
            return carry

        lax.fori_loop(0, NG, grp_body, 0)
        p0 = base + g * P
        pltpu.async_copy(
            outb, out_hbm.at[:, p0 // 128, :, pl.ds(p0 % 128, P)], osem)

    stage_coords(0, cx0, cy0, cz0, xsem0)
    stage_coords(1, cx1, cy1, cz1, xsem1)
    wait_coords(cx0, cy0, cz0, xsem0)
    fire(cx0, cy0, cz0, idx0, w0, rows0, gsem0)

    def body(h, carry):
        g0 = 2 * h
        g1 = g0 + 1
        wait_coords(cx1, cy1, cz1, xsem1)
        fire(cx1, cy1, cz1, idx1, w1, rows1, gsem1)

        @pl.when(g0 + 2 < NCH)
        def _():
            stage_coords(g0 + 2, cx0, cy0, cz0, xsem0)

        combine(g0, w0, rows0, ob0, gsem0, osem0, h >= 1)

        @pl.when(g0 + 2 < NCH)
        def _():
            wait_coords(cx0, cy0, cz0, xsem0)
            fire(cx0, cy0, cz0, idx0, w0, rows0, gsem0)

        @pl.when(g1 + 2 < NCH)
        def _():
            stage_coords(g1 + 2, cx1, cy1, cz1, xsem1)

        combine(g1, w1, rows1, ob1, gsem1, osem1, h >= 1)
        return carry

    lax.fori_loop(0, NCH // 2, body, 0)
    pltpu.make_async_copy(
        out_hbm.at[:, 0, :, pl.ds(0, P)], ob0, osem0).wait()
    pltpu.make_async_copy(
        out_hbm.at[:, 0, :, pl.ds(0, P)], ob1, osem1).wait()


@functools.partial(
    pl.kernel,
    out_type=jax.ShapeDtypeStruct((DIMOUT // 8, N // 128, 8, 128),
                                  jnp.float32),
    mesh=plsc.VectorSubcoreMesh(core_axis_name="c", subcore_axis_name="s"),
    compiler_params=pltpu.CompilerParams(
        use_tc_tiling_on_sc=False, needs_layout_passes=False),
    scratch_types=[
        pltpu.VMEM((P,), jnp.float32),
        pltpu.VMEM((P,), jnp.float32),
        pltpu.VMEM((P,), jnp.float32),
        pltpu.VMEM((P,), jnp.float32),
        pltpu.VMEM((P,), jnp.float32),
        pltpu.VMEM((P,), jnp.float32),
        pltpu.VMEM((NCORNER, P), jnp.int32),
        pltpu.VMEM((NCORNER, P), jnp.int32),
        pltpu.VMEM((NCORNER, P), jnp.float32),
        pltpu.VMEM((NCORNER, P), jnp.float32),
        pltpu.VMEM((NCORNER, P, DIMOUT // 2), jnp.float32),
        pltpu.VMEM((NCORNER, P, DIMOUT // 2), jnp.float32),
        pltpu.VMEM((DIMOUT // 8, 8, P), jnp.float32),
        pltpu.VMEM((DIMOUT // 8, 8, P), jnp.float32),
        pltpu.VMEM((DIMOUT,), jnp.float32),
        pltpu.VMEM((DIMOUT,), jnp.float32),
        pltpu.SemaphoreType.DMA,
        pltpu.SemaphoreType.DMA,
        pltpu.SemaphoreType.DMA,
        pltpu.SemaphoreType.DMA,
        pltpu.SemaphoreType.DMA,
        pltpu.SemaphoreType.DMA,
    ],
)
def _sc_sample(gx_hbm, gy_hbm, gz_hbm, t0_hbm, t1_hbm, t2_hbm, bias_hbm,
               out_hbm, *scratch):
    _sc_body(gx_hbm, gy_hbm, gz_hbm, t0_hbm, t1_hbm, t2_hbm, bias_hbm,
             out_hbm, *scratch)


def kernel(x, xy, xz, yz, lin_w, lin_b):
    gx = x[:, 0]
    gy = x[:, 1]
    gz = x[:, 2]
    wperm = jnp.concatenate([lin_w[0::2, :], lin_w[1::2, :]], axis=0)
    t0, t1, t2 = _transform(xy, xz, yz, wperm)
    t0 = t0.reshape(4 * TBL, DIMOUT // 2)
    t1 = t1.reshape(4 * TBL, DIMOUT // 2)
    t2 = t2.reshape(4 * TBL, DIMOUT // 2)
    out4 = _sc_sample(gx, gy, gz, t0, t1, t2, lin_b)
    return out4.transpose(1, 3, 0, 2).reshape(N, DIMOUT)


# speedup vs baseline: 1.6458x; 1.0052x over previous
import functools

import jax
import jax.numpy as jnp
from jax import lax
from jax.experimental import pallas as pl
from jax.experimental.pallas import tpu as pltpu
from jax.experimental.pallas import tpu_sc as plsc

DIM = 64
DIMOUT = 64
SZ = 512
N = 524288
TBL = SZ * SZ

NC = 2
NS = 16
L = 16
NW = NC * NS
PTS_PER_W = N // NW
P = 128
NCH = PTS_PER_W // P
NG = P // L
NCORNER = 12


_BH = 8


def _transform_body(xy_ref, xz_ref, yz_ref, w_ref, o0, o1, o2):
    for p, (ref, out) in enumerate(((xy_ref, o0), (xz_ref, o1), (yz_ref, o2))):
        we = w_ref[0:DIMOUT // 2, p * DIM:(p + 1) * DIM]
        wo = w_ref[DIMOUT // 2:DIMOUT, p * DIM:(p + 1) * DIM]
        for b in range(_BH):
            blk = ref[:, b, :]
            res_e = lax.dot_general(
                blk, we, (((0,), (1,)), ((), ())),
                preferred_element_type=jnp.float32)
            res_o = lax.dot_general(
                blk, wo, (((0,), (1,)), ((), ())),
                preferred_element_type=jnp.float32)
            ue = lax.bitcast_convert_type(
                res_e.astype(jnp.bfloat16), jnp.uint16).astype(jnp.uint32)
            uo = lax.bitcast_convert_type(
                res_o.astype(jnp.bfloat16), jnp.uint16).astype(jnp.uint32)
            packed = lax.bitcast_convert_type(
                ue | (uo << 16), jnp.float32)
            out[b, :, 0:DIMOUT // 2] = packed


def _transform(xy, xz, yz, lin_w):
    grid = (SZ // _BH,)
    in_spec = pl.BlockSpec((DIM, _BH, SZ), lambda i: (0, i, 0))
    w_spec = pl.BlockSpec((DIMOUT, 3 * DIM), lambda i: (0, 0))
    out_spec = pl.BlockSpec((_BH, SZ, 2 * DIMOUT), lambda i: (i, 0, 0))
    tbl_shape = jax.ShapeDtypeStruct((SZ, SZ, 2 * DIMOUT), jnp.float32)
    return pl.pallas_call(
        _transform_body,
        grid=grid,
        in_specs=[in_spec, in_spec, in_spec, w_spec],
        out_specs=[out_spec, out_spec, out_spec],
        out_shape=[tbl_shape, tbl_shape, tbl_shape],
        compiler_params=pltpu.CompilerParams(
            dimension_semantics=("arbitrary",)),
    )(xy, xz, yz, lin_w)


def _axis_quantities(g):
    ix = ((g + 1.0) * float(SZ) - 1.0) * 0.5
    t = ix.astype(jnp.int32)
    tf = t.astype(jnp.float32)
    x0 = jnp.where(ix < tf, t - 1, t)
    x0f = x0.astype(jnp.float32)
    w1 = ix - x0f
    w0 = 1.0 - w1
    u0 = jnp.where((x0 >= 0) & (x0 <= SZ - 1), w0, 0.0)
    u1 = jnp.where((x0 >= -1) & (x0 <= SZ - 2), w1, 0.0)
    c0 = jnp.clip(x0, 0, SZ - 1)
    c1 = jnp.clip(x0 + 1, 0, SZ - 1)
    return c0, c1, u0, u1


def _sc_body(gx_hbm, gy_hbm, gz_hbm, t0_hbm, t1_hbm, t2_hbm, bias_hbm,
             out_hbm,
             cx0, cy0, cz0, cx1, cy1, cz1, idx0, idx1, w0, w1,
             rows0, rows1, ob0, ob1, biasb, biasp,
             xsem0, xsem1, gsem0, gsem1, osem0, osem1):
    cid = lax.axis_index("c")
    sid = lax.axis_index("s")
    wid = sid * NC + cid
    base = wid * PTS_PER_W
    tables = (t0_hbm, t1_hbm, t2_hbm)

    pltpu.sync_copy(bias_hbm, biasb)
    jv0 = lax.iota(jnp.int32, L)
    for m in range(DIMOUT // L):
        fvec = 32 * (m // 2) + 2 * jv0 + (m % 2)
        biasp[pl.ds(m * L, L)] = plsc.load_gather(biasb, [fvec])

    def stage_coords(g, cx, cy, cz, xsem):
        sl = pl.ds(base + g * P, P)
        pltpu.async_copy(gx_hbm.at[sl], cx, xsem)
        pltpu.async_copy(gy_hbm.at[sl], cy, xsem)
        pltpu.async_copy(gz_hbm.at[sl], cz, xsem)

    def wait_coords(cx, cy, cz, xsem):
        pltpu.make_async_copy(gx_hbm.at[pl.ds(0, P)], cx, xsem).wait()
        pltpu.make_async_copy(gx_hbm.at[pl.ds(0, P)], cy, xsem).wait()
        pltpu.make_async_copy(gx_hbm.at[pl.ds(0, P)], cz, xsem).wait()

    def fire(cx, cy, cz, idx2d, w2d, rows, gsem):
        def grp_body(grp, carry):
            sl = pl.ds(grp * L, L)
            ax = _axis_quantities(cx[sl])
            ay = _axis_quantities(cy[sl])
            az = _axis_quantities(cz[sl])
            c = 0
            for (A, B) in ((ax, ay), (ax, az), (ay, az)):
                b0 = B[0] * (4 * SZ)
                b1 = B[1] * (4 * SZ)
                for (brow, bw) in ((b0, B[2]), (b1, B[3])):
                    for i in (0, 1):
                        idx2d[c, sl] = brow + 4 * A[i]
                        w2d[c, sl] = A[2 + i] * bw
                        c += 1
            return carry

        lax.fori_loop(0, NG, grp_body, 0)
        for c in range(NCORNER):
            pltpu.async_copy(
                tables[c // 4].at[idx2d.at[c]], rows.at[c], gsem)

    def combine(g, w2d, rows, outb, gsem, osem, owait_pred):
        for c in range(NCORNER):
            pltpu.make_async_copy(
                t0_hbm.at[pl.ds(0, P)], rows.at[c], gsem).wait()

        @pl.when(owait_pred)
        def _():
            pltpu.make_async_copy(
                out_hbm.at[:, 0, :, pl.ds(0, P)], outb, osem).wait()

        jv = lax.iota(jnp.int32, L)
        sidx = []
        for h in range(DIMOUT // 32):
            for par in (0, 1):
                f = 32 * h + 2 * jv + par
                sidx.append((lax.shift_right_logical(f, 3),
                             jnp.bitwise_and(f, 7)))

        def grp_body(grp, carry):
            sl = pl.ds(grp * L, L)
            wvs = [w2d[c, sl] for c in range(NCORNER)]
            bias_regs = [biasp[pl.ds(m * L, L)] for m in range(DIMOUT // L)]
            for lane in range(L):
                p = grp * L + lane
                pfull = jnp.full((L,), p, jnp.int32)
                accs = list(bias_regs)
                for c in range(NCORNER):
                    w = wvs[c][lane]
                    for h in range(DIMOUT // 32):
                        half = plsc.bitcast(
                            rows[c, p, pl.ds(h * L, L)], jnp.bfloat16)
                        ev, od = plsc.unpack(
                            half, format=plsc.PackFormat.INTERLEAVED)
                        accs[2 * h] = accs[2 * h] + w * ev
                        accs[2 * h + 1] = accs[2 * h + 1] + w * od
                for a, (bi, fi) in zip(accs, sidx):
                    plsc.store_scatter(outb, [bi, fi, pfull], a)
            return carry

        lax.fori_loop(0, NG, grp_body, 0)
        p0 = base + g * P
        pltpu.async_copy(
            outb, out_hbm.at[:, p0 // 128, :, pl.ds(p0 % 128, P)], osem)

    stage_coords(0, cx0, cy0, cz0, xsem0)
    stage_coords(1, cx1, cy1, cz1, xsem1)
    wait_coords(cx0, cy0, cz0, xsem0)
    fire(cx0, cy0, cz0, idx0, w0, rows0, gsem0)

    def body(h, carry):
        g0 = 2 * h
        g1 = g0 + 1
        wait_coords(cx1, cy1, cz1, xsem1)
        fire(cx1, cy1, cz1, idx1, w1, rows1, gsem1)

        @pl.when(g0 + 2 < NCH)
        def _():
            stage_coords(g0 + 2, cx0, cy0, cz0, xsem0)

        combine(g0, w0, rows0, ob0, gsem0, osem0, h >= 1)

        @pl.when(g0 + 2 < NCH)
        def _():
            wait_coords(cx0, cy0, cz0, xsem0)
            fire(cx0, cy0, cz0, idx0, w0, rows0, gsem0)

        @pl.when(g1 + 2 < NCH)
        def _():
            stage_coords(g1 + 2, cx1, cy1, cz1, xsem1)

        combine(g1, w1, rows1, ob1, gsem1, osem1, h >= 1)
        return carry

    lax.fori_loop(0, NCH // 2, body, 0)
    pltpu.make_async_copy(
        out_hbm.at[:, 0, :, pl.ds(0, P)], ob0, osem0).wait()
    pltpu.make_async_copy(
        out_hbm.at[:, 0, :, pl.ds(0, P)], ob1, osem1).wait()


@functools.partial(
    pl.kernel,
    out_type=jax.ShapeDtypeStruct((DIMOUT // 8, N // 128, 8, 128),
                                  jnp.float32),
    mesh=plsc.VectorSubcoreMesh(core_axis_name="c", subcore_axis_name="s"),
    compiler_params=pltpu.CompilerParams(
        use_tc_tiling_on_sc=False, needs_layout_passes=False),
    scratch_types=[
        pltpu.VMEM((P,), jnp.float32),
        pltpu.VMEM((P,), jnp.float32),
        pltpu.VMEM((P,), jnp.float32),
        pltpu.VMEM((P,), jnp.float32),
        pltpu.VMEM((P,), jnp.float32),
        pltpu.VMEM((P,), jnp.float32),
        pltpu.VMEM((NCORNER, P), jnp.int32),
        pltpu.VMEM((NCORNER, P), jnp.int32),
        pltpu.VMEM((NCORNER, P), jnp.float32),
        pltpu.VMEM((NCORNER, P), jnp.float32),
        pltpu.VMEM((NCORNER, P, DIMOUT // 2), jnp.float32),
        pltpu.VMEM((NCORNER, P, DIMOUT // 2), jnp.float32),
        pltpu.VMEM((DIMOUT // 8, 8, P), jnp.float32),
        pltpu.VMEM((DIMOUT // 8, 8, P), jnp.float32),
        pltpu.VMEM((DIMOUT,), jnp.float32),
        pltpu.VMEM((DIMOUT,), jnp.float32),
        pltpu.SemaphoreType.DMA,
        pltpu.SemaphoreType.DMA,
        pltpu.SemaphoreType.DMA,
        pltpu.SemaphoreType.DMA,
        pltpu.SemaphoreType.DMA,
        pltpu.SemaphoreType.DMA,
    ],
)
def _sc_sample(gx_hbm, gy_hbm, gz_hbm, t0_hbm, t1_hbm, t2_hbm, bias_hbm,
               out_hbm, *scratch):
    _sc_body(gx_hbm, gy_hbm, gz_hbm, t0_hbm, t1_hbm, t2_hbm, bias_hbm,
             out_hbm, *scratch)


def kernel(x, xy, xz, yz, lin_w, lin_b):
    gx = x[:, 0]
    gy = x[:, 1]
    gz = x[:, 2]
    wperm = jnp.concatenate([lin_w[0::2, :], lin_w[1::2, :]], axis=0)
    t0, t1, t2 = _transform(xy, xz, yz, wperm)
    t0 = t0.reshape(4 * TBL, DIMOUT // 2)
    t1 = t1.reshape(4 * TBL, DIMOUT // 2)
    t2 = t2.reshape(4 * TBL, DIMOUT // 2)
    out4 = _sc_sample(gx, gy, gz, t0, t1, t2, lin_b)
    return out4.transpose(1, 3, 0, 2).reshape(N, DIMOUT)
